# baseline (speedup 1.0000x reference)
import functools

import jax
import jax.numpy as jnp
from jax import lax
from jax.experimental import pallas as pl
from jax.experimental.pallas import tpu as pltpu
from jax.experimental.pallas import tpu_sc as plsc

_NUM_CORES = 2
_NUM_SUBCORES = 16
_NW = _NUM_CORES * _NUM_SUBCORES
_LANES = 16
_NBUF = 3
_KB = 8
_X = 512


@functools.partial(jax.jit, static_argnames=("V", "D"))
def _pack_table(wt, *, V, D):
    M = ((V + 3) // 4 + _X - 1) // _X * _X
    ng = M // _X
    lb = (V - 1) // _X

    def body(r0, r1, r2, r3, o_ref):
        x = jnp.concatenate([r0[...], r1[...], r2[...], r3[...]], axis=0)
        eye = jnp.eye(128, dtype=jnp.float32)
        o_ref[...] = jax.lax.dot_general(
            x, eye, (((0,), (0,)), ((), ())),
            preferred_element_type=jnp.float32,
        )

    f = pl.pallas_call(
        body,
        grid=(ng,),
        in_specs=[
            pl.BlockSpec(
                (D, _X), lambda i, a=a: (0, jnp.minimum(a * ng + i, lb))
            )
            for a in range(4)
        ],
        out_specs=pl.BlockSpec((_X, 128), lambda i: (i, 0)),
        out_shape=jax.ShapeDtypeStruct((M, 128), jnp.float32),
    )
    return f(wt, wt, wt, wt)


@functools.partial(jax.jit, static_argnames=("B", "F", "D"))
def _embedding_bag(idx2, weight, *, B, F, D):
    nb = B // _NW
    rpc = 128 // F
    cl = rpc * F
    nch = nb // rpc
    ns = nch // _KB
    rows_ps = _KB * cl
    out_ps = _KB * rpc

    @functools.partial(
        pl.kernel,
        out_type=jax.ShapeDtypeStruct((B, D), jnp.float32),
        mesh=plsc.VectorSubcoreMesh(core_axis_name="c", subcore_axis_name="s"),
        compiler_params=pltpu.CompilerParams(use_tc_tiling_on_sc=False),
        scratch_types=[
            pltpu.VMEM((nch, cl), jnp.int32),
            pltpu.VMEM((nb, D), jnp.float32),
            [pltpu.VMEM((rows_ps, D), jnp.float32) for _ in range(_NBUF)],
            [pltpu.SemaphoreType.DMA for _ in range(_NBUF)],
        ],
    )
    def run(idx_hbm, w_hbm, out_hbm, idx_v, out_v, bufs, sems):
        wid = lax.axis_index("s") * _NUM_CORES + lax.axis_index("c")
        pltpu.sync_copy(idx_hbm.at[pl.ds(wid * nch, nch)], idx_v)

        def fire(s, b):
            return [
                pltpu.async_copy(
                    w_hbm.at[idx_v.at[s * _KB + k]],
                    bufs[b].at[pl.ds(k * cl, cl)],
                    sems[b],
                )
                for k in range(_KB)
            ]

        cps = [fire(s, s % _NBUF) for s in range(_NBUF)]

        for s in range(ns):
            b = s % _NBUF
            for cp in cps[b]:
                cp.wait()
            buf = bufs[b]

            @plsc.parallel_loop(0, out_ps, 1)
            def body(o, buf=buf, s=s):
                m = o * F
                for h in range(0, D, _LANES):
                    pa = buf[m, pl.ds(h, _LANES)]
                    pb = buf[m + 1, pl.ds(h, _LANES)]
                    for j in range(2, F, 2):
                        pa = pa + buf[m + j, pl.ds(h, _LANES)]
                        pb = pb + buf[m + j + 1, pl.ds(h, _LANES)]
                    out_v[s * out_ps + o, pl.ds(h, _LANES)] = pa + pb

            nxt = s + _NBUF
            if nxt < ns:
                cps[b] = fire(nxt, b)

        pltpu.sync_copy(out_v, out_hbm.at[pl.ds(wid * nb, nb)])

    return run(idx2, weight)


def kernel(input, weight):
    B, F = input.shape
    V1, D = weight.shape
    rpc = 128 // F
    M = ((V1 + 3) // 4 + _X - 1) // _X * _X
    w128 = _pack_table(weight.T, V=V1, D=D)
    w2 = w128.reshape(-1, D)
    idx2 = input.reshape(B // rpc, rpc * F)
    a = (idx2 // M).astype(jnp.int32)
    idxL = 4 * (idx2 - a * M) + a
    return _embedding_bag(idxL, w2, B=B, F=F, D=D)


# speedup vs baseline: 1.9689x; 1.9689x over previous
import functools

import jax
import jax.numpy as jnp
from jax import lax
from jax.experimental import pallas as pl
from jax.experimental.pallas import tpu as pltpu
from jax.experimental.pallas import tpu_sc as plsc

_NUM_CORES = 2
_NUM_SUBCORES = 16
_NW = _NUM_CORES * _NUM_SUBCORES
_LANES = 16
_NBUF = 3
_KB = 8
_X = 2048


@functools.partial(jax.jit, static_argnames=("V", "D"))
def _pack_table(wt, *, V, D):
    M = ((V + 3) // 4 + _X - 1) // _X * _X
    ng = M // _X
    lb = (V - 1) // _X

    def body(r0, r1, r2, r3, o_ref):
        x = jnp.concatenate([r0[...], r1[...], r2[...], r3[...]], axis=0)
        o_ref[...] = x.T

    f = pl.pallas_call(
        body,
        grid=(ng,),
        in_specs=[
            pl.BlockSpec(
                (D, _X), lambda i, a=a: (0, jnp.minimum(a * ng + i, lb))
            )
            for a in range(4)
        ],
        out_specs=pl.BlockSpec((_X, 128), lambda i: (i, 0)),
        out_shape=jax.ShapeDtypeStruct((M, 128), jnp.float32),
    )
    return f(wt, wt, wt, wt)


@functools.partial(jax.jit, static_argnames=("B", "F", "D"))
def _embedding_bag(idx2, weight, *, B, F, D):
    nb = B // _NW
    rpc = 128 // F
    cl = rpc * F
    nch = nb // rpc
    ns = nch // _KB
    rows_ps = _KB * cl
    out_ps = _KB * rpc

    @functools.partial(
        pl.kernel,
        out_type=jax.ShapeDtypeStruct((B, D), jnp.float32),
        mesh=plsc.VectorSubcoreMesh(core_axis_name="c", subcore_axis_name="s"),
        compiler_params=pltpu.CompilerParams(use_tc_tiling_on_sc=False),
        scratch_types=[
            pltpu.VMEM((nch, cl), jnp.int32),
            pltpu.VMEM((nb, D), jnp.float32),
            [pltpu.VMEM((rows_ps, D), jnp.float32) for _ in range(_NBUF)],
            [pltpu.SemaphoreType.DMA for _ in range(_NBUF)],
        ],
    )
    def run(idx_hbm, w_hbm, out_hbm, idx_v, out_v, bufs, sems):
        wid = lax.axis_index("s") * _NUM_CORES + lax.axis_index("c")
        pltpu.sync_copy(idx_hbm.at[pl.ds(wid * nch, nch)], idx_v)

        def fire(s, b):
            return [
                pltpu.async_copy(
                    w_hbm.at[idx_v.at[s * _KB + k]],
                    bufs[b].at[pl.ds(k * cl, cl)],
                    sems[b],
                )
                for k in range(_KB)
            ]

        cps = [fire(s, s % _NBUF) for s in range(_NBUF)]

        for s in range(ns):
            b = s % _NBUF
            for cp in cps[b]:
                cp.wait()
            buf = bufs[b]

            @plsc.parallel_loop(0, out_ps, 1)
            def body(o, buf=buf, s=s):
                m = o * F
                for h in range(0, D, _LANES):
                    pa = buf[m, pl.ds(h, _LANES)]
                    pb = buf[m + 1, pl.ds(h, _LANES)]
                    for j in range(2, F, 2):
                        pa = pa + buf[m + j, pl.ds(h, _LANES)]
                        pb = pb + buf[m + j + 1, pl.ds(h, _LANES)]
                    out_v[s * out_ps + o, pl.ds(h, _LANES)] = pa + pb

            nxt = s + _NBUF
            if nxt < ns:
                cps[b] = fire(nxt, b)

        pltpu.sync_copy(out_v, out_hbm.at[pl.ds(wid * nb, nb)])

    return run(idx2, weight)


def kernel(input, weight):
    B, F = input.shape
    V1, D = weight.shape
    rpc = 128 // F
    M = ((V1 + 3) // 4 + _X - 1) // _X * _X
    w128 = _pack_table(weight.T, V=V1, D=D)
    w2 = w128.reshape(-1, D)
    idx2 = input.reshape(B // rpc, rpc * F)
    a = (idx2 // M).astype(jnp.int32)
    idxL = 4 * (idx2 - a * M) + a
    return _embedding_bag(idxL, w2, B=B, F=F, D=D)


# speedup vs baseline: 2.3960x; 1.2169x over previous
import functools

import jax
import jax.numpy as jnp
from jax import lax
from jax.experimental import pallas as pl
from jax.experimental.pallas import tpu as pltpu
from jax.experimental.pallas import tpu_sc as plsc

_NUM_CORES = 2
_NUM_SUBCORES = 16
_NW = _NUM_CORES * _NUM_SUBCORES
_LANES = 16
_NBUF = 3
_KB = 8
_X = 4096


@functools.partial(jax.jit, static_argnames=("V", "D"))
def _pack_table(wt, *, V, D):
    M = ((V + 3) // 4 + _X - 1) // _X * _X
    ng = M // _X
    lb = (V - 1) // _X

    def body(r0, r1, r2, r3, o_ref):
        x = jnp.concatenate([r0[...], r1[...], r2[...], r3[...]], axis=0)
        o_ref[...] = x.T

    f = pl.pallas_call(
        body,
        grid=(ng,),
        in_specs=[
            pl.BlockSpec(
                (D, _X), lambda i, a=a: (0, jnp.minimum(a * ng + i, lb))
            )
            for a in range(4)
        ],
        out_specs=pl.BlockSpec((_X, 128), lambda i: (i, 0)),
        out_shape=jax.ShapeDtypeStruct((M, 128), jnp.float32),
    )
    return f(wt, wt, wt, wt)


@functools.partial(jax.jit, static_argnames=("B", "F", "D"))
def _embedding_bag(idx2, weight, *, B, F, D):
    nb = B // _NW
    rpc = 128 // F
    cl = rpc * F
    nch = nb // rpc
    ns = nch // _KB
    rows_ps = _KB * cl
    out_ps = _KB * rpc

    @functools.partial(
        pl.kernel,
        out_type=jax.ShapeDtypeStruct((B, D), jnp.float32),
        mesh=plsc.VectorSubcoreMesh(core_axis_name="c", subcore_axis_name="s"),
        compiler_params=pltpu.CompilerParams(use_tc_tiling_on_sc=False),
        scratch_types=[
            pltpu.VMEM((nch, cl), jnp.int32),
            pltpu.VMEM((nb, D), jnp.float32),
            [pltpu.VMEM((rows_ps, D), jnp.float32) for _ in range(_NBUF)],
            [pltpu.SemaphoreType.DMA for _ in range(_NBUF)],
        ],
    )
    def run(idx_hbm, w_hbm, out_hbm, idx_v, out_v, bufs, sems):
        wid = lax.axis_index("s") * _NUM_CORES + lax.axis_index("c")
        pltpu.sync_copy(idx_hbm.at[pl.ds(wid * nch, nch)], idx_v)

        def fire(s, b):
            return [
                pltpu.async_copy(
                    w_hbm.at[idx_v.at[s * _KB + k]],
                    bufs[b].at[pl.ds(k * cl, cl)],
                    sems[b],
                )
                for k in range(_KB)
            ]

        cps = [fire(s, s % _NBUF) for s in range(_NBUF)]

        for s in range(ns):
            b = s % _NBUF
            for cp in cps[b]:
                cp.wait()
            buf = bufs[b]

            @plsc.parallel_loop(0, out_ps, 1)
            def body(o, buf=buf, s=s):
                m = o * F
                for h in range(0, D, _LANES):
                    pa = buf[m, pl.ds(h, _LANES)]
                    pb = buf[m + 1, pl.ds(h, _LANES)]
                    for j in range(2, F, 2):
                        pa = pa + buf[m + j, pl.ds(h, _LANES)]
                        pb = pb + buf[m + j + 1, pl.ds(h, _LANES)]
                    out_v[s * out_ps + o, pl.ds(h, _LANES)] = pa + pb

            nxt = s + _NBUF
            if nxt < ns:
                cps[b] = fire(nxt, b)

        pltpu.sync_copy(out_v, out_hbm.at[pl.ds(wid * nb, nb)])

    return run(idx2, weight)


def kernel(input, weight):
    B, F = input.shape
    V1, D = weight.shape
    rpc = 128 // F
    M = ((V1 + 3) // 4 + _X - 1) // _X * _X
    w128 = _pack_table(weight.T, V=V1, D=D)
    w2 = w128.reshape(-1, D)
    idx2 = input.reshape(B // rpc, rpc * F)
    a = (idx2 // M).astype(jnp.int32)
    idxL = 4 * (idx2 - a * M) + a
    return _embedding_bag(idxL, w2, B=B, F=F, D=D)


# speedup vs baseline: 2.6075x; 1.0883x over previous
import functools

import jax
import jax.numpy as jnp
from jax import lax
from jax.experimental import pallas as pl
from jax.experimental.pallas import tpu as pltpu
from jax.experimental.pallas import tpu_sc as plsc

_NUM_CORES = 2
_NUM_SUBCORES = 16
_NW = _NUM_CORES * _NUM_SUBCORES
_LANES = 16
_NBUF = 3
_KB = 8
_X = 8192


@functools.partial(jax.jit, static_argnames=("V", "D"))
def _pack_table(wt, *, V, D):
    M = ((V + 3) // 4 + _X - 1) // _X * _X
    ng = M // _X
    lb = (V - 1) // _X

    def body(r0, r1, r2, r3, o_ref):
        x = jnp.concatenate([r0[...], r1[...], r2[...], r3[...]], axis=0)
        o_ref[...] = x.T

    f = pl.pallas_call(
        body,
        grid=(ng,),
        in_specs=[
            pl.BlockSpec(
                (D, _X), lambda i, a=a: (0, jnp.minimum(a * ng + i, lb))
            )
            for a in range(4)
        ],
        out_specs=pl.BlockSpec((_X, 128), lambda i: (i, 0)),
        out_shape=jax.ShapeDtypeStruct((M, 128), jnp.float32),
    )
    return f(wt, wt, wt, wt)


@functools.partial(jax.jit, static_argnames=("B", "F", "D"))
def _embedding_bag(idx2, weight, *, B, F, D):
    nb = B // _NW
    rpc = 128 // F
    cl = rpc * F
    nch = nb // rpc
    ns = nch // _KB
    rows_ps = _KB * cl
    out_ps = _KB * rpc

    @functools.partial(
        pl.kernel,
        out_type=jax.ShapeDtypeStruct((B, D), jnp.float32),
        mesh=plsc.VectorSubcoreMesh(core_axis_name="c", subcore_axis_name="s"),
        compiler_params=pltpu.CompilerParams(use_tc_tiling_on_sc=False),
        scratch_types=[
            pltpu.VMEM((nch, cl), jnp.int32),
            pltpu.VMEM((nb, D), jnp.float32),
            [pltpu.VMEM((rows_ps, D), jnp.float32) for _ in range(_NBUF)],
            [pltpu.SemaphoreType.DMA for _ in range(_NBUF)],
        ],
    )
    def run(idx_hbm, w_hbm, out_hbm, idx_v, out_v, bufs, sems):
        wid = lax.axis_index("s") * _NUM_CORES + lax.axis_index("c")
        pltpu.sync_copy(idx_hbm.at[pl.ds(wid * nch, nch)], idx_v)

        def fire(s, b):
            return [
                pltpu.async_copy(
                    w_hbm.at[idx_v.at[s * _KB + k]],
                    bufs[b].at[pl.ds(k * cl, cl)],
                    sems[b],
                )
                for k in range(_KB)
            ]

        cps = [fire(s, s % _NBUF) for s in range(_NBUF)]

        for s in range(ns):
            b = s % _NBUF
            for cp in cps[b]:
                cp.wait()
            buf = bufs[b]

            @plsc.parallel_loop(0, out_ps, 1)
            def body(o, buf=buf, s=s):
                m = o * F
                for h in range(0, D, _LANES):
                    pa = buf[m, pl.ds(h, _LANES)]
                    pb = buf[m + 1, pl.ds(h, _LANES)]
                    for j in range(2, F, 2):
                        pa = pa + buf[m + j, pl.ds(h, _LANES)]
                        pb = pb + buf[m + j + 1, pl.ds(h, _LANES)]
                    out_v[s * out_ps + o, pl.ds(h, _LANES)] = pa + pb

            nxt = s + _NBUF
            if nxt < ns:
                cps[b] = fire(nxt, b)

        pltpu.sync_copy(out_v, out_hbm.at[pl.ds(wid * nb, nb)])

    return run(idx2, weight)


def kernel(input, weight):
    B, F = input.shape
    V1, D = weight.shape
    rpc = 128 // F
    M = ((V1 + 3) // 4 + _X - 1) // _X * _X
    w128 = _pack_table(weight.T, V=V1, D=D)
    w2 = w128.reshape(-1, D)
    idx2 = input.reshape(B // rpc, rpc * F)
    a = (idx2 // M).astype(jnp.int32)
    idxL = 4 * (idx2 - a * M) + a
    return _embedding_bag(idxL, w2, B=B, F=F, D=D)


# speedup vs baseline: 2.6364x; 1.0111x over previous
import functools

import jax
import jax.numpy as jnp
from jax import lax
from jax.experimental import pallas as pl
from jax.experimental.pallas import tpu as pltpu
from jax.experimental.pallas import tpu_sc as plsc

_NUM_CORES = 2
_NUM_SUBCORES = 16
_NW = _NUM_CORES * _NUM_SUBCORES
_LANES = 16
_NBUF = 3
_KB = 8
_X = 16384


@functools.partial(jax.jit, static_argnames=("V", "D"))
def _pack_table(wt, *, V, D):
    M = ((V + 3) // 4 + _X - 1) // _X * _X
    ng = M // _X
    lb = (V - 1) // _X

    def body(r0, r1, r2, r3, o_ref):
        x = jnp.concatenate([r0[...], r1[...], r2[...], r3[...]], axis=0)
        o_ref[...] = x.T

    f = pl.pallas_call(
        body,
        grid=(ng,),
        in_specs=[
            pl.BlockSpec(
                (D, _X), lambda i, a=a: (0, jnp.minimum(a * ng + i, lb))
            )
            for a in range(4)
        ],
        out_specs=pl.BlockSpec((_X, 128), lambda i: (i, 0)),
        out_shape=jax.ShapeDtypeStruct((M, 128), jnp.float32),
    )
    return f(wt, wt, wt, wt)


@functools.partial(jax.jit, static_argnames=("B", "F", "D"))
def _embedding_bag(idx2, weight, *, B, F, D):
    nb = B // _NW
    rpc = 128 // F
    cl = rpc * F
    nch = nb // rpc
    ns = nch // _KB
    rows_ps = _KB * cl
    out_ps = _KB * rpc

    @functools.partial(
        pl.kernel,
        out_type=jax.ShapeDtypeStruct((B, D), jnp.float32),
        mesh=plsc.VectorSubcoreMesh(core_axis_name="c", subcore_axis_name="s"),
        compiler_params=pltpu.CompilerParams(use_tc_tiling_on_sc=False),
        scratch_types=[
            pltpu.VMEM((nch, cl), jnp.int32),
            pltpu.VMEM((nb, D), jnp.float32),
            [pltpu.VMEM((rows_ps, D), jnp.float32) for _ in range(_NBUF)],
            [pltpu.SemaphoreType.DMA for _ in range(_NBUF)],
        ],
    )
    def run(idx_hbm, w_hbm, out_hbm, idx_v, out_v, bufs, sems):
        wid = lax.axis_index("s") * _NUM_CORES + lax.axis_index("c")
        pltpu.sync_copy(idx_hbm.at[pl.ds(wid * nch, nch)], idx_v)

        def fire(s, b):
            return [
                pltpu.async_copy(
                    w_hbm.at[idx_v.at[s * _KB + k]],
                    bufs[b].at[pl.ds(k * cl, cl)],
                    sems[b],
                )
                for k in range(_KB)
            ]

        cps = [fire(s, s % _NBUF) for s in range(_NBUF)]

        for s in range(ns):
            b = s % _NBUF
            for cp in cps[b]:
                cp.wait()
            buf = bufs[b]

            @plsc.parallel_loop(0, out_ps, 1)
            def body(o, buf=buf, s=s):
                m = o * F
                for h in range(0, D, _LANES):
                    pa = buf[m, pl.ds(h, _LANES)]
                    pb = buf[m + 1, pl.ds(h, _LANES)]
                    for j in range(2, F, 2):
                        pa = pa + buf[m + j, pl.ds(h, _LANES)]
                        pb = pb + buf[m + j + 1, pl.ds(h, _LANES)]
                    out_v[s * out_ps + o, pl.ds(h, _LANES)] = pa + pb

            nxt = s + _NBUF
            if nxt < ns:
                cps[b] = fire(nxt, b)

        pltpu.sync_copy(out_v, out_hbm.at[pl.ds(wid * nb, nb)])

    return run(idx2, weight)


def kernel(input, weight):
    B, F = input.shape
    V1, D = weight.shape
    rpc = 128 // F
    M = ((V1 + 3) // 4 + _X - 1) // _X * _X
    w128 = _pack_table(weight.T, V=V1, D=D)
    w2 = w128.reshape(-1, D)
    idx2 = input.reshape(B // rpc, rpc * F)
    a = (idx2 // M).astype(jnp.int32)
    idxL = 4 * (idx2 - a * M) + a
    return _embedding_bag(idxL, w2, B=B, F=F, D=D)
